# baseline (speedup 1.0000x reference)
import functools

import jax
import jax.numpy as jnp
from jax import lax
from jax.experimental import pallas as pl
from jax.experimental.pallas import tpu as pltpu
from jax.experimental.pallas import tpu_sc as plsc

_N = 10000
_E = 320000
_D = 128
_NC = 2
_NS = 16
_L = 16
_NW = _NC * _NS
_EW = _E // _NW
_CH = 50
_SC = 8
_NSC = _EW // (_CH * _SC)
_WVP = 512
_EU = 5
_NP = 10240
_RPT = _NP // _NS
_ZR = 40

_BR = 1000
_GRID = _N // _BR


def _sc_aggregate(neigh, row3, col3, w2):
    mesh = plsc.VectorSubcoreMesh(core_axis_name="c", subcore_axis_name="s")

    @functools.partial(
        pl.kernel,
        out_type=jax.ShapeDtypeStruct((_NC, _NP, _D), jnp.float32),
        mesh=mesh,
        compiler_params=pltpu.CompilerParams(needs_layout_passes=False),
        scratch_types=[
            pltpu.VMEM((2 * _SC, _CH), jnp.int32),
            pltpu.VMEM((2 * _SC, _CH), jnp.int32),
            pltpu.VMEM((2, _WVP), jnp.float32),
            pltpu.VMEM((2, _CH, _D), jnp.float32),
            pltpu.VMEM((_ZR, _D), jnp.float32),
            pltpu.VMEM_SHARED((_NP, _D), jnp.float32),
            pltpu.SemaphoreType.DMA,
            pltpu.SemaphoreType.DMA,
            pltpu.SemaphoreType.DMA,
            pltpu.SemaphoreType.DMA,
            pltpu.SemaphoreType.DMA,
            pltpu.SemaphoreType.DMA,
        ],
    )
    def k(neigh_hbm, row_hbm, col_hbm, w_hbm, out_hbm,
          rowv, colv, wv, rows, zbuf, accum,
          gsem0, gsem1, ssem0, ssem1, isem, zsem):
        cid = lax.axis_index("c")
        sid = lax.axis_index("s")
        wid = sid * _NC + cid
        gsem = (gsem0, gsem1)
        ssem = (ssem0, ssem1)

        def zrow(r, carry):
            for u in range(_D // _L):
                zbuf[r, pl.ds(u * _L, _L)] = jnp.zeros((_L,), jnp.float32)
            return carry
        lax.fori_loop(0, _ZR, zrow, 0)

        r1 = rows.at[1]

        def zrow1(r, carry):
            for u in range(_D // _L):
                r1[r, pl.ds(u * _L, _L)] = jnp.zeros((_L,), jnp.float32)
            return carry
        lax.fori_loop(0, _CH, zrow1, 0)
        zcopies = [
            pltpu.async_copy(
                zbuf, accum.at[pl.ds(sid * _RPT + z * _ZR, _ZR)], zsem)
            for z in range(_RPT // _ZR)
        ]
        for c in zcopies:
            c.wait()

        pltpu.sync_copy(row_hbm.at[wid, 0], rowv.at[pl.ds(0, _SC)])
        pltpu.sync_copy(col_hbm.at[wid, 0], colv.at[pl.ds(0, _SC)])
        pltpu.sync_copy(w_hbm.at[wid, 0], wv.at[0])
        pltpu.async_copy(neigh_hbm.at[colv.at[0]], rows.at[0], gsem0)
        plsc.subcore_barrier()
        pltpu.async_copy(rows.at[1], accum.at[rowv.at[1]], ssem1, add=True)

        def superchunk(g, carry):
            p = g % 2
            q = 1 - p
            poff = p * _SC
            qoff = q * _SC
            gn = jnp.minimum(g + 1, _NSC - 1)

            for j in range(_SC):
                b = j % 2
                pltpu.make_async_copy(
                    neigh_hbm.at[colv.at[poff + j]], rows.at[b],
                    gsem[b]).wait()
                pltpu.make_async_copy(
                    rows.at[1 - b], accum.at[rowv.at[0]],
                    ssem[1 - b]).wait()

                if j == 0:
                    pltpu.async_copy(row_hbm.at[wid, gn],
                                     rowv.at[pl.ds(qoff, _SC)], isem)
                    pltpu.async_copy(col_hbm.at[wid, gn],
                                     colv.at[pl.ds(qoff, _SC)], isem)
                    pltpu.async_copy(w_hbm.at[wid, gn], wv.at[q], isem)

                if j < _SC - 1:
                    pltpu.async_copy(neigh_hbm.at[colv.at[poff + j + 1]],
                                     rows.at[1 - b], gsem[1 - b])
                else:
                    pltpu.make_async_copy(
                        row_hbm.at[wid, gn],
                        rowv.at[pl.ds(qoff, _SC)], isem).wait()
                    pltpu.make_async_copy(
                        col_hbm.at[wid, gn],
                        colv.at[pl.ds(qoff, _SC)], isem).wait()
                    pltpu.make_async_copy(
                        w_hbm.at[wid, gn], wv.at[q], isem).wait()
                    pltpu.async_copy(neigh_hbm.at[colv.at[qoff]],
                                     rows.at[0], gsem0)

                rb = rows.at[b]
                pv = jnp.full((_L,), p, jnp.int32)

                def edge(i, c2):
                    e0 = i * _EU
                    wbs = [
                        plsc.load_gather(
                            wv,
                            [pv, jnp.full((_L,), e0 + (j * _CH + k),
                                          jnp.int32)])
                        for k in range(_EU)
                    ]
                    for k in range(_EU):
                        for u in range(_D // _L):
                            s = pl.ds(u * _L, _L)
                            rb[e0 + k, s] = rb[e0 + k, s] * wbs[k]
                    return c2
                lax.fori_loop(0, _CH // _EU, edge, 0)

                pltpu.async_copy(rb, accum.at[rowv.at[poff + j]],
                                 ssem[b], add=True)
            return carry
        lax.fori_loop(0, _NSC, superchunk, 0)

        pltpu.make_async_copy(
            rows.at[1], accum.at[rowv.at[_SC - 1]], ssem1).wait()
        pltpu.make_async_copy(
            neigh_hbm.at[colv.at[_SC]], rows.at[0], gsem0).wait()

        plsc.subcore_barrier()
        pltpu.sync_copy(accum.at[pl.ds(sid * _RPT, _RPT)],
                        out_hbm.at[cid, pl.ds(sid * _RPT, _RPT)])

    return k(neigh, row3, col3, w2)


def _gelu(x):
    return 0.5 * x * (1.0 + lax.erf(x * 0.7071067811865476))


def _wspec(shape):
    n = len(shape)
    return pl.BlockSpec(shape, lambda b, _n=n: (0,) * _n)


def _k_lift(nf, W1, b1, W2, b2, Wn, bn, h_out, ng_out):
    h1 = _gelu(nf[...] @ W1[...].T + b1[...])
    h = h1 @ W2[...].T + b2[...]
    h_out[...] = h
    ng_out[...] = h @ Wn[...].T + bn[...]


def _k_gate_neigh(h, aggr, Ws, bs, gW1s, gW1n, gb1, gW2, gb2, Wn, bn,
                  hn_out, ng_out):
    a = aggr[0] + aggr[1]
    selff = h[...] @ Ws[...].T + bs[...]
    t = selff @ gW1s[...].T + a @ gW1n[...].T + gb1[...]
    g = _gelu(t) @ gW2[...].T + gb2[...]
    hn = h[...] + g
    hn_out[...] = hn
    ng_out[...] = hn @ Wn[...].T + bn[...]


def _k_gate_ln(h, aggr, Ws, bs, gW1s, gW1n, gb1, gW2, gb2, gamma, beta, out):
    a = aggr[0] + aggr[1]
    selff = h[...] @ Ws[...].T + bs[...]
    t = selff @ gW1s[...].T + a @ gW1n[...].T + gb1[...]
    g = _gelu(t) @ gW2[...].T + gb2[...]
    hn = h[...] + g
    mean = jnp.mean(hn, axis=-1, keepdims=True)
    var = jnp.mean((hn - mean) ** 2, axis=-1, keepdims=True)
    out[...] = (hn - mean) / jnp.sqrt(var + 1e-5) * gamma[...] + beta[...]


_ROW_SPEC = pl.BlockSpec((_BR, _D), lambda b: (b, 0))
_AGGR_SPEC = pl.BlockSpec((_NC, _BR, _D), lambda b: (0, b, 0))


def _lift_call(nf, W1, b1, W2, b2, Wn, bn):
    return pl.pallas_call(
        _k_lift,
        grid=(_GRID,),
        in_specs=[
            pl.BlockSpec((_BR, 6), lambda b: (b, 0)),
            _wspec((_D, 6)), _wspec((1, _D)),
            _wspec((_D, _D)), _wspec((1, _D)),
            _wspec((_D, _D)), _wspec((1, _D)),
        ],
        out_specs=[_ROW_SPEC, _ROW_SPEC],
        out_shape=[jax.ShapeDtypeStruct((_N, _D), jnp.float32)] * 2,
    )(nf, W1, b1, W2, b2, Wn, bn)


def _gate_neigh_call(h, aggr, Ws, bs, gW1s, gW1n, gb1, gW2, gb2, Wn, bn):
    return pl.pallas_call(
        _k_gate_neigh,
        grid=(_GRID,),
        in_specs=[
            _ROW_SPEC, _AGGR_SPEC,
            _wspec((_D, _D)), _wspec((1, _D)),
            _wspec((_D, _D)), _wspec((_D, _D)), _wspec((1, _D)),
            _wspec((_D, _D)), _wspec((1, _D)),
            _wspec((_D, _D)), _wspec((1, _D)),
        ],
        out_specs=[_ROW_SPEC, _ROW_SPEC],
        out_shape=[jax.ShapeDtypeStruct((_N, _D), jnp.float32)] * 2,
    )(h, aggr, Ws, bs, gW1s, gW1n, gb1, gW2, gb2, Wn, bn)


def _gate_ln_call(h, aggr, Ws, bs, gW1s, gW1n, gb1, gW2, gb2, gamma, beta):
    return pl.pallas_call(
        _k_gate_ln,
        grid=(_GRID,),
        in_specs=[
            _ROW_SPEC, _AGGR_SPEC,
            _wspec((_D, _D)), _wspec((1, _D)),
            _wspec((_D, _D)), _wspec((_D, _D)), _wspec((1, _D)),
            _wspec((_D, _D)), _wspec((1, _D)),
            _wspec((1, _D)), _wspec((1, _D)),
        ],
        out_specs=_ROW_SPEC,
        out_shape=jax.ShapeDtypeStruct((_N, _D), jnp.float32),
    )(h, aggr, Ws, bs, gW1s, gW1n, gb1, gW2, gb2, gamma, beta)


def kernel(x, edge_index, edge_values,
           lift_W1, lift_b1, lift_W2, lift_b2,
           gcn0_Ws, gcn0_bs, gcn0_Wn, gcn0_bn,
           gcn0_gW1, gcn0_gb1, gcn0_gW2, gcn0_gb2,
           gcn1_Ws, gcn1_bs, gcn1_Wn, gcn1_bn,
           gcn1_gW1, gcn1_gb1, gcn1_gW2, gcn1_gb2,
           ln_gamma, ln_beta):
    nf = x[0, :, 3:]
    row3 = edge_index[0].reshape(_NW, _NSC, _SC, _CH)
    col3 = edge_index[1].reshape(_NW, _NSC, _SC, _CH)
    w2 = jnp.pad(edge_values.reshape(_NW, _NSC, _SC * _CH),
                 ((0, 0), (0, 0), (0, _WVP - _SC * _CH)))

    r = lambda v: v.reshape(1, _D)
    g0s, g0n = gcn0_gW1[:, :_D], gcn0_gW1[:, _D:]
    g1s, g1n = gcn1_gW1[:, :_D], gcn1_gW1[:, _D:]

    h, ng = _lift_call(nf, lift_W1, r(lift_b1), lift_W2, r(lift_b2),
                       gcn0_Wn, r(gcn0_bn))
    aggr0 = _sc_aggregate(ng, row3, col3, w2)
    h, ng = _gate_neigh_call(h, aggr0, gcn0_Ws, r(gcn0_bs),
                             g0s, g0n, r(gcn0_gb1), gcn0_gW2, r(gcn0_gb2),
                             gcn1_Wn, r(gcn1_bn))
    aggr1 = _sc_aggregate(ng, row3, col3, w2)
    out = _gate_ln_call(h, aggr1, gcn1_Ws, r(gcn1_bs),
                        g1s, g1n, r(gcn1_gb1), gcn1_gW2, r(gcn1_gb2),
                        r(ln_gamma), r(ln_beta))
    return out[None, :, :]


# speedup vs baseline: 1.3348x; 1.3348x over previous
import functools

import jax
import jax.numpy as jnp
from jax import lax
from jax.experimental import pallas as pl
from jax.experimental.pallas import tpu as pltpu
from jax.experimental.pallas import tpu_sc as plsc

_N = 10000
_E = 320000
_D = 128
_NC = 2
_NS = 16
_L = 16
_NW = _NC * _NS
_EW = _E // _NW
_CH = 100
_SC = 4
_NSC = _EW // (_CH * _SC)
_WVP = 512
_EU = 5
_NP = 10240
_RPT = _NP // _NS
_ZR = 40

_BR = 1000
_GRID = _N // _BR


def _sc_aggregate(neigh, row3, col3, w2):
    mesh = plsc.VectorSubcoreMesh(core_axis_name="c", subcore_axis_name="s")

    @functools.partial(
        pl.kernel,
        out_type=jax.ShapeDtypeStruct((_NC, _NP, _D), jnp.float32),
        mesh=mesh,
        compiler_params=pltpu.CompilerParams(needs_layout_passes=False),
        scratch_types=[
            pltpu.VMEM((2 * _SC, _CH), jnp.int32),
            pltpu.VMEM((2 * _SC, _CH), jnp.int32),
            pltpu.VMEM((2, _WVP), jnp.float32),
            pltpu.VMEM((2, _CH, _D), jnp.float32),
            pltpu.VMEM((_ZR, _D), jnp.float32),
            pltpu.VMEM_SHARED((_NP, _D), jnp.float32),
            pltpu.SemaphoreType.DMA,
            pltpu.SemaphoreType.DMA,
            pltpu.SemaphoreType.DMA,
            pltpu.SemaphoreType.DMA,
            pltpu.SemaphoreType.DMA,
            pltpu.SemaphoreType.DMA,
        ],
    )
    def k(neigh_hbm, row_hbm, col_hbm, w_hbm, out_hbm,
          rowv, colv, wv, rows, zbuf, accum,
          gsem0, gsem1, ssem0, ssem1, isem, zsem):
        cid = lax.axis_index("c")
        sid = lax.axis_index("s")
        wid = sid * _NC + cid
        gsem = (gsem0, gsem1)
        ssem = (ssem0, ssem1)

        def zrow(r, carry):
            for u in range(_D // _L):
                zbuf[r, pl.ds(u * _L, _L)] = jnp.zeros((_L,), jnp.float32)
            return carry
        lax.fori_loop(0, _ZR, zrow, 0)

        r1 = rows.at[1]

        def zrow1(r, carry):
            for u in range(_D // _L):
                r1[r, pl.ds(u * _L, _L)] = jnp.zeros((_L,), jnp.float32)
            return carry
        lax.fori_loop(0, _CH, zrow1, 0)
        zcopies = [
            pltpu.async_copy(
                zbuf, accum.at[pl.ds(sid * _RPT + z * _ZR, _ZR)], zsem)
            for z in range(_RPT // _ZR)
        ]
        for c in zcopies:
            c.wait()

        pltpu.sync_copy(row_hbm.at[wid, 0], rowv.at[pl.ds(0, _SC)])
        pltpu.sync_copy(col_hbm.at[wid, 0], colv.at[pl.ds(0, _SC)])
        pltpu.sync_copy(w_hbm.at[wid, 0], wv.at[0])
        pltpu.async_copy(neigh_hbm.at[colv.at[0]], rows.at[0], gsem0)
        plsc.subcore_barrier()
        pltpu.async_copy(rows.at[1], accum.at[rowv.at[1]], ssem1, add=True)

        def superchunk(g, carry):
            p = g % 2
            q = 1 - p
            poff = p * _SC
            qoff = q * _SC
            gn = jnp.minimum(g + 1, _NSC - 1)

            for j in range(_SC):
                b = j % 2
                pltpu.make_async_copy(
                    neigh_hbm.at[colv.at[poff + j]], rows.at[b],
                    gsem[b]).wait()
                pltpu.make_async_copy(
                    rows.at[1 - b], accum.at[rowv.at[0]],
                    ssem[1 - b]).wait()

                if j == 0:
                    pltpu.async_copy(row_hbm.at[wid, gn],
                                     rowv.at[pl.ds(qoff, _SC)], isem)
                    pltpu.async_copy(col_hbm.at[wid, gn],
                                     colv.at[pl.ds(qoff, _SC)], isem)
                    pltpu.async_copy(w_hbm.at[wid, gn], wv.at[q], isem)

                if j < _SC - 1:
                    pltpu.async_copy(neigh_hbm.at[colv.at[poff + j + 1]],
                                     rows.at[1 - b], gsem[1 - b])
                else:
                    pltpu.make_async_copy(
                        row_hbm.at[wid, gn],
                        rowv.at[pl.ds(qoff, _SC)], isem).wait()
                    pltpu.make_async_copy(
                        col_hbm.at[wid, gn],
                        colv.at[pl.ds(qoff, _SC)], isem).wait()
                    pltpu.make_async_copy(
                        w_hbm.at[wid, gn], wv.at[q], isem).wait()
                    pltpu.async_copy(neigh_hbm.at[colv.at[qoff]],
                                     rows.at[0], gsem0)

                rb = rows.at[b]
                pv = jnp.full((_L,), p, jnp.int32)

                def edge(i, c2):
                    e0 = i * _EU
                    wbs = [
                        plsc.load_gather(
                            wv,
                            [pv, jnp.full((_L,), e0 + (j * _CH + k),
                                          jnp.int32)])
                        for k in range(_EU)
                    ]
                    for k in range(_EU):
                        for u in range(_D // _L):
                            s = pl.ds(u * _L, _L)
                            rb[e0 + k, s] = rb[e0 + k, s] * wbs[k]
                    return c2
                lax.fori_loop(0, _CH // _EU, edge, 0)

                pltpu.async_copy(rb, accum.at[rowv.at[poff + j]],
                                 ssem[b], add=True)
            return carry
        lax.fori_loop(0, _NSC, superchunk, 0)

        pltpu.make_async_copy(
            rows.at[1], accum.at[rowv.at[_SC - 1]], ssem1).wait()
        pltpu.make_async_copy(
            neigh_hbm.at[colv.at[_SC]], rows.at[0], gsem0).wait()

        plsc.subcore_barrier()
        pltpu.sync_copy(accum.at[pl.ds(sid * _RPT, _RPT)],
                        out_hbm.at[cid, pl.ds(sid * _RPT, _RPT)])

    return k(neigh, row3, col3, w2)


def _gelu(x):
    return 0.5 * x * (1.0 + lax.erf(x * 0.7071067811865476))


def _wspec(shape):
    n = len(shape)
    return pl.BlockSpec(shape, lambda b, _n=n: (0,) * _n)


def _k_lift(nf, W1, b1, W2, b2, Wn, bn, h_out, ng_out):
    h1 = _gelu(nf[...] @ W1[...].T + b1[...])
    h = h1 @ W2[...].T + b2[...]
    h_out[...] = h
    ng_out[...] = h @ Wn[...].T + bn[...]


def _k_gate_neigh(h, aggr, Ws, bs, gW1s, gW1n, gb1, gW2, gb2, Wn, bn,
                  hn_out, ng_out):
    a = aggr[0] + aggr[1]
    selff = h[...] @ Ws[...].T + bs[...]
    t = selff @ gW1s[...].T + a @ gW1n[...].T + gb1[...]
    g = _gelu(t) @ gW2[...].T + gb2[...]
    hn = h[...] + g
    hn_out[...] = hn
    ng_out[...] = hn @ Wn[...].T + bn[...]


def _k_gate_ln(h, aggr, Ws, bs, gW1s, gW1n, gb1, gW2, gb2, gamma, beta, out):
    a = aggr[0] + aggr[1]
    selff = h[...] @ Ws[...].T + bs[...]
    t = selff @ gW1s[...].T + a @ gW1n[...].T + gb1[...]
    g = _gelu(t) @ gW2[...].T + gb2[...]
    hn = h[...] + g
    mean = jnp.mean(hn, axis=-1, keepdims=True)
    var = jnp.mean((hn - mean) ** 2, axis=-1, keepdims=True)
    out[...] = (hn - mean) / jnp.sqrt(var + 1e-5) * gamma[...] + beta[...]


_ROW_SPEC = pl.BlockSpec((_BR, _D), lambda b: (b, 0))
_AGGR_SPEC = pl.BlockSpec((_NC, _BR, _D), lambda b: (0, b, 0))


def _lift_call(nf, W1, b1, W2, b2, Wn, bn):
    return pl.pallas_call(
        _k_lift,
        grid=(_GRID,),
        in_specs=[
            pl.BlockSpec((_BR, 6), lambda b: (b, 0)),
            _wspec((_D, 6)), _wspec((1, _D)),
            _wspec((_D, _D)), _wspec((1, _D)),
            _wspec((_D, _D)), _wspec((1, _D)),
        ],
        out_specs=[_ROW_SPEC, _ROW_SPEC],
        out_shape=[jax.ShapeDtypeStruct((_N, _D), jnp.float32)] * 2,
    )(nf, W1, b1, W2, b2, Wn, bn)


def _gate_neigh_call(h, aggr, Ws, bs, gW1s, gW1n, gb1, gW2, gb2, Wn, bn):
    return pl.pallas_call(
        _k_gate_neigh,
        grid=(_GRID,),
        in_specs=[
            _ROW_SPEC, _AGGR_SPEC,
            _wspec((_D, _D)), _wspec((1, _D)),
            _wspec((_D, _D)), _wspec((_D, _D)), _wspec((1, _D)),
            _wspec((_D, _D)), _wspec((1, _D)),
            _wspec((_D, _D)), _wspec((1, _D)),
        ],
        out_specs=[_ROW_SPEC, _ROW_SPEC],
        out_shape=[jax.ShapeDtypeStruct((_N, _D), jnp.float32)] * 2,
    )(h, aggr, Ws, bs, gW1s, gW1n, gb1, gW2, gb2, Wn, bn)


def _gate_ln_call(h, aggr, Ws, bs, gW1s, gW1n, gb1, gW2, gb2, gamma, beta):
    return pl.pallas_call(
        _k_gate_ln,
        grid=(_GRID,),
        in_specs=[
            _ROW_SPEC, _AGGR_SPEC,
            _wspec((_D, _D)), _wspec((1, _D)),
            _wspec((_D, _D)), _wspec((_D, _D)), _wspec((1, _D)),
            _wspec((_D, _D)), _wspec((1, _D)),
            _wspec((1, _D)), _wspec((1, _D)),
        ],
        out_specs=_ROW_SPEC,
        out_shape=jax.ShapeDtypeStruct((_N, _D), jnp.float32),
    )(h, aggr, Ws, bs, gW1s, gW1n, gb1, gW2, gb2, gamma, beta)


def kernel(x, edge_index, edge_values,
           lift_W1, lift_b1, lift_W2, lift_b2,
           gcn0_Ws, gcn0_bs, gcn0_Wn, gcn0_bn,
           gcn0_gW1, gcn0_gb1, gcn0_gW2, gcn0_gb2,
           gcn1_Ws, gcn1_bs, gcn1_Wn, gcn1_bn,
           gcn1_gW1, gcn1_gb1, gcn1_gW2, gcn1_gb2,
           ln_gamma, ln_beta):
    nf = x[0, :, 3:]
    row3 = edge_index[0].reshape(_NW, _NSC, _SC, _CH)
    col3 = edge_index[1].reshape(_NW, _NSC, _SC, _CH)
    w2 = jnp.pad(edge_values.reshape(_NW, _NSC, _SC * _CH),
                 ((0, 0), (0, 0), (0, _WVP - _SC * _CH)))

    r = lambda v: v.reshape(1, _D)
    g0s, g0n = gcn0_gW1[:, :_D], gcn0_gW1[:, _D:]
    g1s, g1n = gcn1_gW1[:, :_D], gcn1_gW1[:, _D:]

    h, ng = _lift_call(nf, lift_W1, r(lift_b1), lift_W2, r(lift_b2),
                       gcn0_Wn, r(gcn0_bn))
    aggr0 = _sc_aggregate(ng, row3, col3, w2)
    h, ng = _gate_neigh_call(h, aggr0, gcn0_Ws, r(gcn0_bs),
                             g0s, g0n, r(gcn0_gb1), gcn0_gW2, r(gcn0_gb2),
                             gcn1_Wn, r(gcn1_bn))
    aggr1 = _sc_aggregate(ng, row3, col3, w2)
    out = _gate_ln_call(h, aggr1, gcn1_Ws, r(gcn1_bs),
                        g1s, g1n, r(gcn1_gb1), gcn1_gW2, r(gcn1_gb2),
                        r(ln_gamma), r(ln_beta))
    return out[None, :, :]


# speedup vs baseline: 1.3725x; 1.0282x over previous
import functools

import jax
import jax.numpy as jnp
from jax import lax
from jax.experimental import pallas as pl
from jax.experimental.pallas import tpu as pltpu
from jax.experimental.pallas import tpu_sc as plsc

_N = 10000
_E = 320000
_D = 128
_NC = 2
_NS = 16
_L = 16
_NW = _NC * _NS
_EW = _E // _NW
_CH = 125
_SC = 4
_NSC = _EW // (_CH * _SC)
_WVP = 512
_EU = 5
_NP = 10240
_RPT = _NP // _NS
_ZR = 40

_BR = 1000
_GRID = _N // _BR


def _sc_aggregate(neigh, row3, col3, w2):
    mesh = plsc.VectorSubcoreMesh(core_axis_name="c", subcore_axis_name="s")

    @functools.partial(
        pl.kernel,
        out_type=jax.ShapeDtypeStruct((_NC, _NP, _D), jnp.float32),
        mesh=mesh,
        compiler_params=pltpu.CompilerParams(needs_layout_passes=False),
        scratch_types=[
            pltpu.VMEM((2 * _SC, _CH), jnp.int32),
            pltpu.VMEM((2 * _SC, _CH), jnp.int32),
            pltpu.VMEM((2, _WVP), jnp.float32),
            pltpu.VMEM((2, _CH, _D), jnp.float32),
            pltpu.VMEM((_ZR, _D), jnp.float32),
            pltpu.VMEM_SHARED((_NP, _D), jnp.float32),
            pltpu.SemaphoreType.DMA,
            pltpu.SemaphoreType.DMA,
            pltpu.SemaphoreType.DMA,
            pltpu.SemaphoreType.DMA,
            pltpu.SemaphoreType.DMA,
            pltpu.SemaphoreType.DMA,
        ],
    )
    def k(neigh_hbm, row_hbm, col_hbm, w_hbm, out_hbm,
          rowv, colv, wv, rows, zbuf, accum,
          gsem0, gsem1, ssem0, ssem1, isem, zsem):
        cid = lax.axis_index("c")
        sid = lax.axis_index("s")
        wid = sid * _NC + cid
        gsem = (gsem0, gsem1)
        ssem = (ssem0, ssem1)

        def zrow(r, carry):
            for u in range(_D // _L):
                zbuf[r, pl.ds(u * _L, _L)] = jnp.zeros((_L,), jnp.float32)
            return carry
        lax.fori_loop(0, _ZR, zrow, 0)

        r1 = rows.at[1]

        def zrow1(r, carry):
            for u in range(_D // _L):
                r1[r, pl.ds(u * _L, _L)] = jnp.zeros((_L,), jnp.float32)
            return carry
        lax.fori_loop(0, _CH, zrow1, 0)
        zcopies = [
            pltpu.async_copy(
                zbuf, accum.at[pl.ds(sid * _RPT + z * _ZR, _ZR)], zsem)
            for z in range(_RPT // _ZR)
        ]
        for c in zcopies:
            c.wait()

        pltpu.sync_copy(row_hbm.at[wid, 0], rowv.at[pl.ds(0, _SC)])
        pltpu.sync_copy(col_hbm.at[wid, 0], colv.at[pl.ds(0, _SC)])
        pltpu.sync_copy(w_hbm.at[wid, 0], wv.at[0])
        pltpu.async_copy(neigh_hbm.at[colv.at[0]], rows.at[0], gsem0)
        plsc.subcore_barrier()
        pltpu.async_copy(rows.at[1], accum.at[rowv.at[1]], ssem1, add=True)

        def superchunk(g, carry):
            p = g % 2
            q = 1 - p
            poff = p * _SC
            qoff = q * _SC
            gn = jnp.minimum(g + 1, _NSC - 1)

            for j in range(_SC):
                b = j % 2
                pltpu.make_async_copy(
                    neigh_hbm.at[colv.at[poff + j]], rows.at[b],
                    gsem[b]).wait()
                pltpu.make_async_copy(
                    rows.at[1 - b], accum.at[rowv.at[0]],
                    ssem[1 - b]).wait()

                if j == 0:
                    pltpu.async_copy(row_hbm.at[wid, gn],
                                     rowv.at[pl.ds(qoff, _SC)], isem)
                    pltpu.async_copy(col_hbm.at[wid, gn],
                                     colv.at[pl.ds(qoff, _SC)], isem)
                    pltpu.async_copy(w_hbm.at[wid, gn], wv.at[q], isem)

                if j < _SC - 1:
                    pltpu.async_copy(neigh_hbm.at[colv.at[poff + j + 1]],
                                     rows.at[1 - b], gsem[1 - b])
                else:
                    pltpu.make_async_copy(
                        row_hbm.at[wid, gn],
                        rowv.at[pl.ds(qoff, _SC)], isem).wait()
                    pltpu.make_async_copy(
                        col_hbm.at[wid, gn],
                        colv.at[pl.ds(qoff, _SC)], isem).wait()
                    pltpu.make_async_copy(
                        w_hbm.at[wid, gn], wv.at[q], isem).wait()
                    pltpu.async_copy(neigh_hbm.at[colv.at[qoff]],
                                     rows.at[0], gsem0)

                rb = rows.at[b]
                pv = jnp.full((_L,), p, jnp.int32)

                def edge(i, c2):
                    e0 = i * _EU
                    wbs = [
                        plsc.load_gather(
                            wv,
                            [pv, jnp.full((_L,), e0 + (j * _CH + k),
                                          jnp.int32)])
                        for k in range(_EU)
                    ]
                    for k in range(_EU):
                        for u in range(_D // _L):
                            s = pl.ds(u * _L, _L)
                            rb[e0 + k, s] = rb[e0 + k, s] * wbs[k]
                    return c2
                lax.fori_loop(0, _CH // _EU, edge, 0)

                pltpu.async_copy(rb, accum.at[rowv.at[poff + j]],
                                 ssem[b], add=True)
            return carry
        lax.fori_loop(0, _NSC, superchunk, 0)

        pltpu.make_async_copy(
            rows.at[1], accum.at[rowv.at[_SC - 1]], ssem1).wait()
        pltpu.make_async_copy(
            neigh_hbm.at[colv.at[_SC]], rows.at[0], gsem0).wait()

        plsc.subcore_barrier()
        pltpu.sync_copy(accum.at[pl.ds(sid * _RPT, _RPT)],
                        out_hbm.at[cid, pl.ds(sid * _RPT, _RPT)])

    return k(neigh, row3, col3, w2)


def _gelu(x):
    return 0.5 * x * (1.0 + lax.erf(x * 0.7071067811865476))


def _wspec(shape):
    n = len(shape)
    return pl.BlockSpec(shape, lambda b, _n=n: (0,) * _n)


def _k_lift(nf, W1, b1, W2, b2, Wn, bn, h_out, ng_out):
    h1 = _gelu(nf[...] @ W1[...].T + b1[...])
    h = h1 @ W2[...].T + b2[...]
    h_out[...] = h
    ng_out[...] = h @ Wn[...].T + bn[...]


def _k_gate_neigh(h, aggr, Ws, bs, gW1s, gW1n, gb1, gW2, gb2, Wn, bn,
                  hn_out, ng_out):
    a = aggr[0] + aggr[1]
    selff = h[...] @ Ws[...].T + bs[...]
    t = selff @ gW1s[...].T + a @ gW1n[...].T + gb1[...]
    g = _gelu(t) @ gW2[...].T + gb2[...]
    hn = h[...] + g
    hn_out[...] = hn
    ng_out[...] = hn @ Wn[...].T + bn[...]


def _k_gate_ln(h, aggr, Ws, bs, gW1s, gW1n, gb1, gW2, gb2, gamma, beta, out):
    a = aggr[0] + aggr[1]
    selff = h[...] @ Ws[...].T + bs[...]
    t = selff @ gW1s[...].T + a @ gW1n[...].T + gb1[...]
    g = _gelu(t) @ gW2[...].T + gb2[...]
    hn = h[...] + g
    mean = jnp.mean(hn, axis=-1, keepdims=True)
    var = jnp.mean((hn - mean) ** 2, axis=-1, keepdims=True)
    out[...] = (hn - mean) / jnp.sqrt(var + 1e-5) * gamma[...] + beta[...]


_ROW_SPEC = pl.BlockSpec((_BR, _D), lambda b: (b, 0))
_AGGR_SPEC = pl.BlockSpec((_NC, _BR, _D), lambda b: (0, b, 0))


def _lift_call(nf, W1, b1, W2, b2, Wn, bn):
    return pl.pallas_call(
        _k_lift,
        grid=(_GRID,),
        in_specs=[
            pl.BlockSpec((_BR, 6), lambda b: (b, 0)),
            _wspec((_D, 6)), _wspec((1, _D)),
            _wspec((_D, _D)), _wspec((1, _D)),
            _wspec((_D, _D)), _wspec((1, _D)),
        ],
        out_specs=[_ROW_SPEC, _ROW_SPEC],
        out_shape=[jax.ShapeDtypeStruct((_N, _D), jnp.float32)] * 2,
    )(nf, W1, b1, W2, b2, Wn, bn)


def _gate_neigh_call(h, aggr, Ws, bs, gW1s, gW1n, gb1, gW2, gb2, Wn, bn):
    return pl.pallas_call(
        _k_gate_neigh,
        grid=(_GRID,),
        in_specs=[
            _ROW_SPEC, _AGGR_SPEC,
            _wspec((_D, _D)), _wspec((1, _D)),
            _wspec((_D, _D)), _wspec((_D, _D)), _wspec((1, _D)),
            _wspec((_D, _D)), _wspec((1, _D)),
            _wspec((_D, _D)), _wspec((1, _D)),
        ],
        out_specs=[_ROW_SPEC, _ROW_SPEC],
        out_shape=[jax.ShapeDtypeStruct((_N, _D), jnp.float32)] * 2,
    )(h, aggr, Ws, bs, gW1s, gW1n, gb1, gW2, gb2, Wn, bn)


def _gate_ln_call(h, aggr, Ws, bs, gW1s, gW1n, gb1, gW2, gb2, gamma, beta):
    return pl.pallas_call(
        _k_gate_ln,
        grid=(_GRID,),
        in_specs=[
            _ROW_SPEC, _AGGR_SPEC,
            _wspec((_D, _D)), _wspec((1, _D)),
            _wspec((_D, _D)), _wspec((_D, _D)), _wspec((1, _D)),
            _wspec((_D, _D)), _wspec((1, _D)),
            _wspec((1, _D)), _wspec((1, _D)),
        ],
        out_specs=_ROW_SPEC,
        out_shape=jax.ShapeDtypeStruct((_N, _D), jnp.float32),
    )(h, aggr, Ws, bs, gW1s, gW1n, gb1, gW2, gb2, gamma, beta)


def kernel(x, edge_index, edge_values,
           lift_W1, lift_b1, lift_W2, lift_b2,
           gcn0_Ws, gcn0_bs, gcn0_Wn, gcn0_bn,
           gcn0_gW1, gcn0_gb1, gcn0_gW2, gcn0_gb2,
           gcn1_Ws, gcn1_bs, gcn1_Wn, gcn1_bn,
           gcn1_gW1, gcn1_gb1, gcn1_gW2, gcn1_gb2,
           ln_gamma, ln_beta):
    nf = x[0, :, 3:]
    row3 = edge_index[0].reshape(_NW, _NSC, _SC, _CH)
    col3 = edge_index[1].reshape(_NW, _NSC, _SC, _CH)
    w2 = jnp.pad(edge_values.reshape(_NW, _NSC, _SC * _CH),
                 ((0, 0), (0, 0), (0, _WVP - _SC * _CH)))

    r = lambda v: v.reshape(1, _D)
    g0s, g0n = gcn0_gW1[:, :_D], gcn0_gW1[:, _D:]
    g1s, g1n = gcn1_gW1[:, :_D], gcn1_gW1[:, _D:]

    h, ng = _lift_call(nf, lift_W1, r(lift_b1), lift_W2, r(lift_b2),
                       gcn0_Wn, r(gcn0_bn))
    aggr0 = _sc_aggregate(ng, row3, col3, w2)
    h, ng = _gate_neigh_call(h, aggr0, gcn0_Ws, r(gcn0_bs),
                             g0s, g0n, r(gcn0_gb1), gcn0_gW2, r(gcn0_gb2),
                             gcn1_Wn, r(gcn1_bn))
    aggr1 = _sc_aggregate(ng, row3, col3, w2)
    out = _gate_ln_call(h, aggr1, gcn1_Ws, r(gcn1_bs),
                        g1s, g1n, r(gcn1_gb1), gcn1_gW2, r(gcn1_gb2),
                        r(ln_gamma), r(ln_beta))
    return out[None, :, :]
